# baseline (speedup 1.0000x reference)
import functools

import jax
import jax.numpy as jnp
import numpy as np
from jax import lax
from jax.experimental import pallas as pl
from jax.experimental.pallas import tpu as pltpu
from jax.experimental.pallas import tpu_sc as plsc

B = 4
C = 256
L = 512
E = 1024
HID = 32
CAP = 4
NSLOT = E * B * CAP
TRASH = NSLOT
XROWS = NSLOT + 8

_INV_SQRT2 = np.float32(1.0 / np.sqrt(2.0))
_ATT_SCALE = np.float32(1.0 / np.sqrt(32.0))


def _gelu(v):
    return 0.5 * v * (1.0 + lax.erf(v * _INV_SQRT2))


def _dot_hi(a, b):
    return jnp.dot(a, b, preferred_element_type=jnp.float32,
                   precision=lax.Precision.HIGHEST)


def _dot_d(a, b):
    return jnp.dot(a, b, preferred_element_type=jnp.float32)


def _group_norm(x, gamma, beta):
    b, c, l = x.shape
    xr = x.reshape(b, 32, c // 32, l)
    mean = xr.mean(axis=(2, 3), keepdims=True)
    var = xr.var(axis=(2, 3), keepdims=True)
    xr = (xr - mean) / jnp.sqrt(var + 1e-5)
    x = xr.reshape(b, c, l)
    return x * gamma[None, :, None] + beta[None, :, None]


def _conv_body(x_ref, w_ref, b_ref, out_ref):
    x2d = x_ref[0]
    nc = x2d.shape[0]
    zc = jnp.zeros((nc, 1), jnp.float32)
    xr = jnp.concatenate([zc, x2d[:, :-1]], axis=1)
    xl = jnp.concatenate([x2d[:, 1:], zc], axis=1)
    out_ref[0] = (_dot_d(w_ref[0], xr) + _dot_d(w_ref[1], x2d)
                  + _dot_d(w_ref[2], xl) + b_ref[...])


def _run_conv(x, w, b):
    co = w.shape[0]
    ci = w.shape[1]
    w3 = jnp.moveaxis(w, 2, 0)
    return pl.pallas_call(
        _conv_body,
        grid=(B,),
        in_specs=[pl.BlockSpec((1, ci, L), lambda i: (i, 0, 0)),
                  pl.BlockSpec((3, co, ci), lambda i: (0, 0, 0)),
                  pl.BlockSpec((co, 1), lambda i: (0, 0))],
        out_specs=pl.BlockSpec((1, co, L), lambda i: (i, 0, 0)),
        out_shape=jax.ShapeDtypeStruct((B, co, L), jnp.float32),
    )(x, w3, b.reshape(co, 1))


def _attn_body(x_ref, aw1_ref, ab1_ref, aw2_ref, ab2_ref, out_ref):
    y = x_ref[0].T
    qkv = _dot_d(y, aw1_ref[...]) + ab1_ref[...]
    outs = []
    for h in range(8):
        q = qkv[:, h * 32:h * 32 + 32]
        k = qkv[:, 256 + h * 32:256 + h * 32 + 32]
        v = qkv[:, 512 + h * 32:512 + h * 32 + 32]
        s = _dot_d(q, k.T) * _ATT_SCALE
        s = s - s.max(axis=-1, keepdims=True)
        e = jnp.exp(s)
        outs.append(_dot_d(e, v) / e.sum(axis=-1, keepdims=True))
    iota_r = lax.broadcasted_iota(jnp.int32, (32, C), 0)
    iota_c = lax.broadcasted_iota(jnp.int32, (32, C), 1)
    o = jnp.zeros((L, C), jnp.float32)
    for h in range(8):
        sel = ((iota_c % 8 == h) & (iota_c // 8 == iota_r)).astype(jnp.float32)
        o = o + _dot_hi(outs[h], sel)
    o = _dot_d(o, aw2_ref[...]) + ab2_ref[...]
    out_ref[0] = o.T


def _run_attn(x, pa):
    w1p = pa['w1'].reshape(32, 8, 3, C).transpose(2, 1, 0, 3).reshape(768, C).T
    b1p = pa['b1'].reshape(32, 8, 3).transpose(2, 1, 0).reshape(1, 768)
    w2t = pa['w2'].T
    b2p = pa['b2'].reshape(1, C)
    return pl.pallas_call(
        _attn_body,
        grid=(B,),
        in_specs=[pl.BlockSpec((1, C, L), lambda i: (i, 0, 0)),
                  pl.BlockSpec((C, 768), lambda i: (0, 0)),
                  pl.BlockSpec((1, 768), lambda i: (0, 0)),
                  pl.BlockSpec((C, C), lambda i: (0, 0)),
                  pl.BlockSpec((1, C), lambda i: (0, 0))],
        out_specs=pl.BlockSpec((1, C, L), lambda i: (i, 0, 0)),
        out_shape=jax.ShapeDtypeStruct((B, C, L), jnp.float32),
    )(x, w1p, b1p, w2t, b2p)


def _gelu_x(v):
    return jax.nn.gelu(v, approximate=False)


def _res_block(x, emb, p):
    x = x + emb
    r = _run_conv(_gelu_x(_group_norm(x, p['gn1_w'], p['gn1_b'])),
                  p['conv1_w'], p['conv1_b'])
    r = _run_conv(_gelu_x(_group_norm(r, p['gn2_w'], p['gn2_b'])),
                  p['conv2_w'], p['conv2_b'])
    return r + x


def _run_front(x, emb, params):
    x = _res_block(x, emb, params['rb1'])
    x = _run_attn(x, params['attn'])
    x = _res_block(x, emb, params['rb2'])
    return x


def _gating_body(x_ref, wg_ref, s1_ref, s2_ref, sg1_ref, sg2_ref,
                 g1_ref, g2_ref, loss_ref):
    iota_e = lax.broadcasted_iota(jnp.int32, (C, E), 1)
    ltri = (lax.broadcasted_iota(jnp.int32, (C, C), 0)
            >= lax.broadcasted_iota(jnp.int32, (C, C), 1)).astype(jnp.float32)
    capf = np.float32(CAP)
    loss_acc = jnp.zeros((1, 1), jnp.float32)
    for b in range(B):
        logits = _dot_d(x_ref[b], wg_ref[...])
        m = logits.max(axis=-1, keepdims=True)
        ex = jnp.exp(logits - m)
        raw = ex / ex.sum(axis=-1, keepdims=True)
        gate1 = raw.max(axis=-1, keepdims=True)
        idx1 = jnp.where(raw >= gate1, iota_e, E).min(axis=-1, keepdims=True)
        mask1 = (iota_e == idx1).astype(jnp.float32)
        gwo = raw * (1.0 - mask1)
        gate2 = gwo.max(axis=-1, keepdims=True)
        idx2 = jnp.where(gwo >= gate2, iota_e, E).min(axis=-1, keepdims=True)
        mask2 = (iota_e == idx2).astype(jnp.float32)
        denom = gate1 + gate2 + 1e-9
        g1n = gate1 / denom
        g2n = gate2 / denom
        density1 = mask1.sum(axis=0, keepdims=True) * np.float32(1.0 / C)
        proxy = raw.sum(axis=0, keepdims=True) * np.float32(1.0 / C)
        loss_acc = loss_acc + jnp.dot(proxy, density1.T,
                                      preferred_element_type=jnp.float32)
        cum1 = jnp.dot(ltri, mask1, preferred_element_type=jnp.float32)
        pos1m = (cum1 - mask1) * mask1
        pos1 = pos1m.sum(axis=-1, keepdims=True)
        mask1c = mask1 * (pos1m < capf).astype(jnp.float32)
        m1count = mask1c.sum(axis=0, keepdims=True)
        m1flat = mask1c.sum(axis=-1, keepdims=True)
        g1 = g1n * m1flat
        cum2 = jnp.dot(ltri, mask2, preferred_element_type=jnp.float32)
        pos2m = ((cum2 - mask2) + m1count) * mask2
        mask2c = mask2 * (pos2m < capf).astype(jnp.float32)
        m2flat = mask2c.sum(axis=-1, keepdims=True)
        pos2 = pos2m.sum(axis=-1, keepdims=True)
        g2 = g2n * m2flat
        s1 = idx1 * (B * CAP) + (b * CAP) + pos1.astype(jnp.int32)
        s2 = idx2 * (B * CAP) + (b * CAP) + pos2.astype(jnp.int32)
        v1 = g1 != 0.0
        v2 = g2 != 0.0
        s1v = jnp.where(v1, s1, TRASH)
        s2v = jnp.where(v2, s2, TRASH)
        s1_ref[b] = s1v.reshape(1, C)
        s2_ref[b] = s2v.reshape(1, C)
        sg1_ref[b] = jnp.minimum(s1v, NSLOT - 1).reshape(1, C)
        sg2_ref[b] = jnp.minimum(s2v, NSLOT - 1).reshape(1, C)
        g1_ref[b] = g1.reshape(1, C)
        g2_ref[b] = g2.reshape(1, C)
    loss_ref[...] = loss_acc * np.float32(float(E * E) / (B * E) * 0.01)


def _run_gating(x2, w_gating):
    outs = pl.pallas_call(
        _gating_body,
        out_shape=(jax.ShapeDtypeStruct((B, 1, C), jnp.int32),
                   jax.ShapeDtypeStruct((B, 1, C), jnp.int32),
                   jax.ShapeDtypeStruct((B, 1, C), jnp.int32),
                   jax.ShapeDtypeStruct((B, 1, C), jnp.int32),
                   jax.ShapeDtypeStruct((B, 1, C), jnp.float32),
                   jax.ShapeDtypeStruct((B, 1, C), jnp.float32),
                   jax.ShapeDtypeStruct((1, 1), jnp.float32)),
    )(x2, w_gating)
    return outs


def _sc_info():
    info = plsc.get_sparse_core_info()
    return info.num_cores, info.num_subcores


def _make_sc_scatter():
    nc, ns = _sc_info()
    nw = nc * ns
    rows_per = (B * C) // nw
    mesh = plsc.VectorSubcoreMesh(core_axis_name="c", subcore_axis_name="s")

    @functools.partial(
        pl.kernel, mesh=mesh,
        out_type=jax.ShapeDtypeStruct((XROWS, C), jnp.float32),
        scratch_types=[pltpu.VMEM((rows_per,), jnp.int32),
                       pltpu.VMEM((rows_per,), jnp.int32),
                       pltpu.VMEM((rows_per, C), jnp.float32),
                       pltpu.SemaphoreType.DMA],
    )
    def scatter_k(x_hbm, s1_hbm, s2_hbm, out_hbm, idx1_v, idx2_v, rows_v, sem):
        wid = lax.axis_index("s") * nc + lax.axis_index("c")
        base = wid * rows_per
        pltpu.sync_copy(s1_hbm.at[pl.ds(base, rows_per)], idx1_v)
        pltpu.sync_copy(s2_hbm.at[pl.ds(base, rows_per)], idx2_v)
        pltpu.sync_copy(x_hbm.at[pl.ds(base, rows_per)], rows_v)
        pltpu.async_copy(rows_v, out_hbm.at[idx1_v], sem).wait()
        pltpu.async_copy(rows_v, out_hbm.at[idx2_v], sem).wait()

    return scatter_k


def _make_sc_gather():
    nc, ns = _sc_info()
    nw = nc * ns
    rows_per = (B * C) // nw
    mesh = plsc.VectorSubcoreMesh(core_axis_name="c", subcore_axis_name="s")

    @functools.partial(
        pl.kernel, mesh=mesh,
        out_type=(jax.ShapeDtypeStruct((B * C, C), jnp.float32),
                  jax.ShapeDtypeStruct((B * C, C), jnp.float32)),
        scratch_types=[pltpu.VMEM((rows_per,), jnp.int32),
                       pltpu.VMEM((rows_per,), jnp.int32),
                       pltpu.VMEM((rows_per, C), jnp.float32),
                       pltpu.VMEM((rows_per, C), jnp.float32),
                       pltpu.SemaphoreType.DMA],
    )
    def gather_k(eo_hbm, s1_hbm, s2_hbm, a_hbm, b_hbm,
                 idx1_v, idx2_v, rows1_v, rows2_v, sem):
        wid = lax.axis_index("s") * nc + lax.axis_index("c")
        base = wid * rows_per
        pltpu.sync_copy(s1_hbm.at[pl.ds(base, rows_per)], idx1_v)
        pltpu.sync_copy(s2_hbm.at[pl.ds(base, rows_per)], idx2_v)
        pltpu.async_copy(eo_hbm.at[idx1_v], rows1_v, sem).wait()
        pltpu.async_copy(eo_hbm.at[idx2_v], rows2_v, sem).wait()
        pltpu.sync_copy(rows1_v, a_hbm.at[pl.ds(base, rows_per)])
        pltpu.sync_copy(rows2_v, b_hbm.at[pl.ds(base, rows_per)])

    return gather_k


ECHUNK = 128


def _experts_body(x_ref, w1_ref, w2_ref, out_ref):
    xb = x_ref[...]
    h = lax.dot_general(xb, w1_ref[...],
                        (((2,), (2,)), ((0,), (0,))),
                        preferred_element_type=jnp.float32)
    h = _gelu(h).astype(jnp.bfloat16)
    o = lax.dot_general(h, w2_ref[...],
                        (((2,), (1,)), ((0,), (0,))),
                        preferred_element_type=jnp.float32)
    out_ref[...] = o.astype(jnp.bfloat16)


def _run_experts(xe_bf, w1t_bf, w2_bf):
    return pl.pallas_call(
        _experts_body,
        grid=(E // ECHUNK,),
        in_specs=[pl.BlockSpec((ECHUNK, B * CAP, L), lambda i: (i, 0, 0)),
                  pl.BlockSpec((ECHUNK, HID, L), lambda i: (i, 0, 0)),
                  pl.BlockSpec((ECHUNK, HID, L), lambda i: (i, 0, 0))],
        out_specs=pl.BlockSpec((ECHUNK, B * CAP, L), lambda i: (i, 0, 0)),
        out_shape=jax.ShapeDtypeStruct((E, B * CAP, L), jnp.bfloat16),
    )(xe_bf, w1t_bf, w2_bf)


def _tail_body(a_ref, b_ref, g1_ref, g2_ref, cw_ref, cb_ref, y_ref, xm_ref):
    a = a_ref[0].astype(jnp.float32)
    bv = b_ref[0].astype(jnp.float32)
    g1 = g1_ref[0].T.astype(jnp.bfloat16).astype(jnp.float32)
    g2 = g2_ref[0].T.astype(jnp.bfloat16).astype(jnp.float32)
    xm = (jnp.where(g1 != 0.0, g1 * a, 0.0)
          + jnp.where(g2 != 0.0, g2 * bv, 0.0))
    xm_ref[0] = xm

    xb = xm.astype(jnp.bfloat16)
    zc = jnp.zeros((C, 1), jnp.bfloat16)
    xr = jnp.concatenate([zc, xb[:, :-1]], axis=1)
    xl = jnp.concatenate([xb[:, 1:], zc], axis=1)
    yc = (jnp.dot(cw_ref[0], xr, preferred_element_type=jnp.float32)
          + jnp.dot(cw_ref[1], xb, preferred_element_type=jnp.float32)
          + jnp.dot(cw_ref[2], xl, preferred_element_type=jnp.float32)
          + cb_ref[...])
    neg = np.float32(-np.inf)
    ninf = jnp.full((2 * C, 1), neg, jnp.float32)
    ycr = jnp.concatenate([ninf, yc[:, :-1]], axis=1)
    ycl = jnp.concatenate([yc[:, 1:], ninf], axis=1)
    wfull = jnp.maximum(jnp.maximum(ycr, yc), ycl)
    sel = (lax.broadcasted_iota(jnp.int32, (L, C), 0)
           == 2 * lax.broadcasted_iota(jnp.int32, (L, C), 1)).astype(jnp.float32)
    y_ref[0] = _dot_hi(wfull, sel)


def _run_tail(a_bf, b_bf, g1, g2, conv_w3, conv_b):
    full = lambda s: pl.BlockSpec(s, lambda i: (0,) * len(s))
    return pl.pallas_call(
        _tail_body,
        grid=(B,),
        in_specs=[pl.BlockSpec((1, C, L), lambda i: (i, 0, 0)),
                  pl.BlockSpec((1, C, L), lambda i: (i, 0, 0)),
                  pl.BlockSpec((1, 1, C), lambda i: (i, 0, 0)),
                  pl.BlockSpec((1, 1, C), lambda i: (i, 0, 0)),
                  full((3, 2 * C, C)),
                  full((2 * C, 1))],
        out_specs=(pl.BlockSpec((1, 2 * C, C), lambda i: (i, 0, 0)),
                   pl.BlockSpec((1, C, L), lambda i: (i, 0, 0))),
        out_shape=(jax.ShapeDtypeStruct((B, 2 * C, C), jnp.float32),
                   jax.ShapeDtypeStruct((B, C, L), jnp.float32)),
    )(a_bf, b_bf, g1, g2, conv_w3, conv_b)


def _pack_bf16_rows(x_bf):
    shp = x_bf.shape
    return lax.bitcast_convert_type(
        x_bf.reshape(shp[:-1] + (shp[-1] // 2, 2)), jnp.float32)


def _unpack_bf16_rows(x_f32):
    shp = x_f32.shape
    return lax.bitcast_convert_type(x_f32, jnp.bfloat16).reshape(
        shp[:-1] + (shp[-1] * 2,))


def kernel(x, embeddings, params):
    x2 = _run_front(x, embeddings, params)

    s1, s2, sg1, sg2, g1, g2, loss = _run_gating(x2, params['w_gating'])
    aux = loss.reshape(()) if loss.shape != () else loss

    x2_bf = x2.astype(jnp.bfloat16).reshape(B * C, L)
    x2_pk = _pack_bf16_rows(x2_bf)
    scatter_k = _make_sc_scatter()
    xe_pk = scatter_k(x2_pk, s1.reshape(B * C), s2.reshape(B * C))

    xe_bf = _unpack_bf16_rows(xe_pk[:NSLOT]).reshape(E, B * CAP, L)
    w1t_bf = jnp.swapaxes(params['moe_w1'], 1, 2).astype(jnp.bfloat16)
    w2_bf = params['moe_w2'].astype(jnp.bfloat16)
    eo_bf = _run_experts(xe_bf, w1t_bf, w2_bf)

    eo_pk = _pack_bf16_rows(eo_bf.reshape(NSLOT, L))
    gather_k = _make_sc_gather()
    a_pk, b_pk = gather_k(eo_pk, sg1.reshape(B * C), sg2.reshape(B * C))
    a_bf = _unpack_bf16_rows(a_pk).reshape(B, C, L)
    b_bf = _unpack_bf16_rows(b_pk).reshape(B, C, L)

    conv_w3 = jnp.moveaxis(params['conv_w'], 2, 0).astype(jnp.bfloat16)
    conv_b = params['conv_b'].reshape(2 * C, 1)
    y, xm = _run_tail(a_bf, b_bf, g1, g2, conv_w3, conv_b)
    return (y, xm, aux.reshape(()))


# speedup vs baseline: 2.0804x; 2.0804x over previous
import functools

import jax
import jax.numpy as jnp
import numpy as np
from jax import lax
from jax.experimental import pallas as pl
from jax.experimental.pallas import tpu as pltpu
from jax.experimental.pallas import tpu_sc as plsc

B = 4
C = 256
L = 512
E = 1024
HID = 32
CAP = 4
NSLOT = E * B * CAP
TRASH = NSLOT
XROWS = NSLOT + 8

_INV_SQRT2 = np.float32(1.0 / np.sqrt(2.0))
_ATT_SCALE = np.float32(1.0 / np.sqrt(32.0))


def _gelu(v):
    return 0.5 * v * (1.0 + lax.erf(v * _INV_SQRT2))


def _dot_hi(a, b):
    return jnp.dot(a, b, preferred_element_type=jnp.float32,
                   precision=lax.Precision.HIGHEST)


def _dot_d(a, b):
    return jnp.dot(a, b, preferred_element_type=jnp.float32)


def _group_norm(x, gamma, beta):
    b, c, l = x.shape
    xr = x.reshape(b, 32, c // 32, l)
    mean = xr.mean(axis=(2, 3), keepdims=True)
    var = xr.var(axis=(2, 3), keepdims=True)
    xr = (xr - mean) / jnp.sqrt(var + 1e-5)
    x = xr.reshape(b, c, l)
    return x * gamma[None, :, None] + beta[None, :, None]


def _conv_body(x_ref, w_ref, b_ref, out_ref):
    x2d = x_ref[0]
    nc = x2d.shape[0]
    zc = jnp.zeros((nc, 1), jnp.float32)
    xr = jnp.concatenate([zc, x2d[:, :-1]], axis=1)
    xl = jnp.concatenate([x2d[:, 1:], zc], axis=1)
    out_ref[0] = (_dot_d(w_ref[0], xr) + _dot_d(w_ref[1], x2d)
                  + _dot_d(w_ref[2], xl) + b_ref[...])


def _run_conv(x, w, b):
    co = w.shape[0]
    ci = w.shape[1]
    w3 = jnp.moveaxis(w, 2, 0)
    return pl.pallas_call(
        _conv_body,
        grid=(B,),
        in_specs=[pl.BlockSpec((1, ci, L), lambda i: (i, 0, 0)),
                  pl.BlockSpec((3, co, ci), lambda i: (0, 0, 0)),
                  pl.BlockSpec((co, 1), lambda i: (0, 0))],
        out_specs=pl.BlockSpec((1, co, L), lambda i: (i, 0, 0)),
        out_shape=jax.ShapeDtypeStruct((B, co, L), jnp.float32),
    )(x, w3, b.reshape(co, 1))


def _attn_body(x_ref, aw1_ref, ab1_ref, aw2_ref, ab2_ref, out_ref):
    y = x_ref[0].T
    qkv = _dot_d(y, aw1_ref[...]) + ab1_ref[...]
    outs = []
    for h in range(8):
        q = qkv[:, h * 32:h * 32 + 32]
        k = qkv[:, 256 + h * 32:256 + h * 32 + 32]
        v = qkv[:, 512 + h * 32:512 + h * 32 + 32]
        s = _dot_d(q, k.T) * _ATT_SCALE
        s = s - s.max(axis=-1, keepdims=True)
        e = jnp.exp(s)
        outs.append(_dot_d(e, v) / e.sum(axis=-1, keepdims=True))
    iota_r = lax.broadcasted_iota(jnp.int32, (32, C), 0)
    iota_c = lax.broadcasted_iota(jnp.int32, (32, C), 1)
    o = jnp.zeros((L, C), jnp.float32)
    for h in range(8):
        sel = ((iota_c % 8 == h) & (iota_c // 8 == iota_r)).astype(jnp.float32)
        o = o + _dot_hi(outs[h], sel)
    o = _dot_d(o, aw2_ref[...]) + ab2_ref[...]
    out_ref[0] = o.T


def _run_attn(x, pa):
    w1p = pa['w1'].reshape(32, 8, 3, C).transpose(2, 1, 0, 3).reshape(768, C).T
    b1p = pa['b1'].reshape(32, 8, 3).transpose(2, 1, 0).reshape(1, 768)
    w2t = pa['w2'].T
    b2p = pa['b2'].reshape(1, C)
    return pl.pallas_call(
        _attn_body,
        grid=(B,),
        in_specs=[pl.BlockSpec((1, C, L), lambda i: (i, 0, 0)),
                  pl.BlockSpec((C, 768), lambda i: (0, 0)),
                  pl.BlockSpec((1, 768), lambda i: (0, 0)),
                  pl.BlockSpec((C, C), lambda i: (0, 0)),
                  pl.BlockSpec((1, C), lambda i: (0, 0))],
        out_specs=pl.BlockSpec((1, C, L), lambda i: (i, 0, 0)),
        out_shape=jax.ShapeDtypeStruct((B, C, L), jnp.float32),
    )(x, w1p, b1p, w2t, b2p)


def _gelu_x(v):
    return jax.nn.gelu(v, approximate=False)


def _res_block(x, emb, p):
    x = x + emb
    r = _run_conv(_gelu_x(_group_norm(x, p['gn1_w'], p['gn1_b'])),
                  p['conv1_w'], p['conv1_b'])
    r = _run_conv(_gelu_x(_group_norm(r, p['gn2_w'], p['gn2_b'])),
                  p['conv2_w'], p['conv2_b'])
    return r + x


def _run_front(x, emb, params):
    x = _res_block(x, emb, params['rb1'])
    x = _run_attn(x, params['attn'])
    x = _res_block(x, emb, params['rb2'])
    return x


def _gating_body(x_ref, wg_ref, s1_ref, s2_ref, sg1_ref, sg2_ref,
                 g1_ref, g2_ref, loss_ref):
    iota_e = lax.broadcasted_iota(jnp.int32, (C, E), 1)
    ltri = (lax.broadcasted_iota(jnp.int32, (C, C), 0)
            >= lax.broadcasted_iota(jnp.int32, (C, C), 1)).astype(jnp.float32)
    capf = np.float32(CAP)
    loss_acc = jnp.zeros((1, 1), jnp.float32)
    for b in range(B):
        logits = _dot_d(x_ref[b], wg_ref[...])
        m = logits.max(axis=-1, keepdims=True)
        ex = jnp.exp(logits - m)
        raw = ex / ex.sum(axis=-1, keepdims=True)
        gate1 = raw.max(axis=-1, keepdims=True)
        idx1 = jnp.where(raw >= gate1, iota_e, E).min(axis=-1, keepdims=True)
        mask1 = (iota_e == idx1).astype(jnp.float32)
        gwo = raw * (1.0 - mask1)
        gate2 = gwo.max(axis=-1, keepdims=True)
        idx2 = jnp.where(gwo >= gate2, iota_e, E).min(axis=-1, keepdims=True)
        mask2 = (iota_e == idx2).astype(jnp.float32)
        denom = gate1 + gate2 + 1e-9
        g1n = gate1 / denom
        g2n = gate2 / denom
        density1 = mask1.sum(axis=0, keepdims=True) * np.float32(1.0 / C)
        proxy = raw.sum(axis=0, keepdims=True) * np.float32(1.0 / C)
        loss_acc = loss_acc + jnp.dot(proxy, density1.T,
                                      preferred_element_type=jnp.float32)
        cum1 = jnp.dot(ltri, mask1, preferred_element_type=jnp.float32)
        pos1m = (cum1 - mask1) * mask1
        pos1 = pos1m.sum(axis=-1, keepdims=True)
        mask1c = mask1 * (pos1m < capf).astype(jnp.float32)
        m1count = mask1c.sum(axis=0, keepdims=True)
        m1flat = mask1c.sum(axis=-1, keepdims=True)
        g1 = g1n * m1flat
        cum2 = jnp.dot(ltri, mask2, preferred_element_type=jnp.float32)
        pos2m = ((cum2 - mask2) + m1count) * mask2
        mask2c = mask2 * (pos2m < capf).astype(jnp.float32)
        m2flat = mask2c.sum(axis=-1, keepdims=True)
        pos2 = pos2m.sum(axis=-1, keepdims=True)
        g2 = g2n * m2flat
        s1 = idx1 * (B * CAP) + (b * CAP) + pos1.astype(jnp.int32)
        s2 = idx2 * (B * CAP) + (b * CAP) + pos2.astype(jnp.int32)
        v1 = g1 != 0.0
        v2 = g2 != 0.0
        s1v = jnp.where(v1, s1, TRASH)
        s2v = jnp.where(v2, s2, TRASH)
        s1_ref[b] = s1v.reshape(1, C)
        s2_ref[b] = s2v.reshape(1, C)
        sg1_ref[b] = jnp.minimum(s1v, NSLOT - 1).reshape(1, C)
        sg2_ref[b] = jnp.minimum(s2v, NSLOT - 1).reshape(1, C)
        g1_ref[b] = g1.reshape(1, C)
        g2_ref[b] = g2.reshape(1, C)
    loss_ref[...] = loss_acc * np.float32(float(E * E) / (B * E) * 0.01)


def _run_gating(x2, w_gating):
    outs = pl.pallas_call(
        _gating_body,
        out_shape=(jax.ShapeDtypeStruct((B, 1, C), jnp.int32),
                   jax.ShapeDtypeStruct((B, 1, C), jnp.int32),
                   jax.ShapeDtypeStruct((B, 1, C), jnp.int32),
                   jax.ShapeDtypeStruct((B, 1, C), jnp.int32),
                   jax.ShapeDtypeStruct((B, 1, C), jnp.float32),
                   jax.ShapeDtypeStruct((B, 1, C), jnp.float32),
                   jax.ShapeDtypeStruct((1, 1), jnp.float32)),
    )(x2, w_gating)
    return outs


def _sc_info():
    info = plsc.get_sparse_core_info()
    return info.num_cores, info.num_subcores


def _make_sc_scatter():
    nc, ns = _sc_info()
    nw = nc * ns
    rows_per = (B * C) // nw
    mesh = plsc.VectorSubcoreMesh(core_axis_name="c", subcore_axis_name="s")

    @functools.partial(
        pl.kernel, mesh=mesh,
        out_type=jax.ShapeDtypeStruct((XROWS, L), jnp.float32),
        scratch_types=[pltpu.VMEM((rows_per,), jnp.int32),
                       pltpu.VMEM((rows_per,), jnp.int32),
                       pltpu.VMEM((rows_per, L), jnp.float32),
                       pltpu.SemaphoreType.DMA],
    )
    def scatter_k(x_hbm, s1_hbm, s2_hbm, out_hbm, idx1_v, idx2_v, rows_v, sem):
        wid = lax.axis_index("s") * nc + lax.axis_index("c")
        base = wid * rows_per
        pltpu.sync_copy(s1_hbm.at[pl.ds(base, rows_per)], idx1_v)
        pltpu.sync_copy(s2_hbm.at[pl.ds(base, rows_per)], idx2_v)
        pltpu.sync_copy(x_hbm.at[pl.ds(base, rows_per)], rows_v)
        pltpu.async_copy(rows_v, out_hbm.at[idx1_v], sem).wait()
        pltpu.async_copy(rows_v, out_hbm.at[idx2_v], sem).wait()

    return scatter_k


def _make_sc_gather():
    nc, ns = _sc_info()
    nw = nc * ns
    rows_per = (B * C) // nw
    mesh = plsc.VectorSubcoreMesh(core_axis_name="c", subcore_axis_name="s")

    @functools.partial(
        pl.kernel, mesh=mesh,
        out_type=(jax.ShapeDtypeStruct((B * C, L), jnp.float32),
                  jax.ShapeDtypeStruct((B * C, L), jnp.float32)),
        scratch_types=[pltpu.VMEM((rows_per,), jnp.int32),
                       pltpu.VMEM((rows_per,), jnp.int32),
                       pltpu.VMEM((rows_per, L), jnp.float32),
                       pltpu.VMEM((rows_per, L), jnp.float32),
                       pltpu.SemaphoreType.DMA],
    )
    def gather_k(eo_hbm, s1_hbm, s2_hbm, a_hbm, b_hbm,
                 idx1_v, idx2_v, rows1_v, rows2_v, sem):
        wid = lax.axis_index("s") * nc + lax.axis_index("c")
        base = wid * rows_per
        pltpu.sync_copy(s1_hbm.at[pl.ds(base, rows_per)], idx1_v)
        pltpu.sync_copy(s2_hbm.at[pl.ds(base, rows_per)], idx2_v)
        pltpu.async_copy(eo_hbm.at[idx1_v], rows1_v, sem).wait()
        pltpu.async_copy(eo_hbm.at[idx2_v], rows2_v, sem).wait()
        pltpu.sync_copy(rows1_v, a_hbm.at[pl.ds(base, rows_per)])
        pltpu.sync_copy(rows2_v, b_hbm.at[pl.ds(base, rows_per)])

    return gather_k


ECHUNK = 32


def _experts_body(x_ref, w1_ref, w2_ref, out_ref):
    xb = x_ref[...].reshape(ECHUNK, B * CAP, L)
    h = lax.dot_general(xb, w1_ref[...],
                        (((2,), (1,)), ((0,), (0,))),
                        preferred_element_type=jnp.float32)
    h = _gelu(h)
    o = lax.dot_general(h, w2_ref[...],
                        (((2,), (1,)), ((0,), (0,))),
                        preferred_element_type=jnp.float32)
    out_ref[...] = o.reshape(ECHUNK * B * CAP, L)


def _run_experts(xe, w1, w2):
    return pl.pallas_call(
        _experts_body,
        grid=(E // ECHUNK,),
        in_specs=[pl.BlockSpec((ECHUNK * B * CAP, L), lambda i: (i, 0)),
                  pl.BlockSpec((ECHUNK, L, HID), lambda i: (i, 0, 0)),
                  pl.BlockSpec((ECHUNK, HID, L), lambda i: (i, 0, 0))],
        out_specs=pl.BlockSpec((ECHUNK * B * CAP, L), lambda i: (i, 0)),
        out_shape=jax.ShapeDtypeStruct((NSLOT, L), jnp.float32),
    )(xe, w1, w2)


def _tail_body(a_ref, b_ref, g1_ref, g2_ref, cw_ref, cb_ref, y_ref, xm_ref):
    a = a_ref[0].astype(jnp.bfloat16).astype(jnp.float32)
    bv = b_ref[0].astype(jnp.bfloat16).astype(jnp.float32)
    g1 = g1_ref[0].T.astype(jnp.bfloat16).astype(jnp.float32)
    g2 = g2_ref[0].T.astype(jnp.bfloat16).astype(jnp.float32)
    xm = (jnp.where(g1 != 0.0, g1 * a, 0.0)
          + jnp.where(g2 != 0.0, g2 * bv, 0.0))
    xm_ref[0] = xm

    xb = xm
    zc = jnp.zeros((C, 1), jnp.float32)
    xr = jnp.concatenate([zc, xb[:, :-1]], axis=1)
    xl = jnp.concatenate([xb[:, 1:], zc], axis=1)
    yc = (jnp.dot(cw_ref[0], xr, preferred_element_type=jnp.float32)
          + jnp.dot(cw_ref[1], xb, preferred_element_type=jnp.float32)
          + jnp.dot(cw_ref[2], xl, preferred_element_type=jnp.float32)
          + cb_ref[...])
    neg = np.float32(-np.inf)
    ninf = jnp.full((2 * C, 1), neg, jnp.float32)
    ycr = jnp.concatenate([ninf, yc[:, :-1]], axis=1)
    ycl = jnp.concatenate([yc[:, 1:], ninf], axis=1)
    wfull = jnp.maximum(jnp.maximum(ycr, yc), ycl)
    sel = (lax.broadcasted_iota(jnp.int32, (L, C), 0)
           == 2 * lax.broadcasted_iota(jnp.int32, (L, C), 1)).astype(jnp.float32)
    y_ref[0] = _dot_hi(wfull, sel)


def _run_tail(a_f, b_f, g1, g2, conv_w3, conv_b):
    full = lambda s: pl.BlockSpec(s, lambda i: (0,) * len(s))
    return pl.pallas_call(
        _tail_body,
        grid=(B,),
        in_specs=[pl.BlockSpec((1, C, L), lambda i: (i, 0, 0)),
                  pl.BlockSpec((1, C, L), lambda i: (i, 0, 0)),
                  pl.BlockSpec((1, 1, C), lambda i: (i, 0, 0)),
                  pl.BlockSpec((1, 1, C), lambda i: (i, 0, 0)),
                  full((3, 2 * C, C)),
                  full((2 * C, 1))],
        out_specs=(pl.BlockSpec((1, 2 * C, C), lambda i: (i, 0, 0)),
                   pl.BlockSpec((1, C, L), lambda i: (i, 0, 0))),
        out_shape=(jax.ShapeDtypeStruct((B, 2 * C, C), jnp.float32),
                   jax.ShapeDtypeStruct((B, C, L), jnp.float32)),
    )(a_f, b_f, g1, g2, conv_w3, conv_b)


def _pack_bf16_rows(x_bf):
    shp = x_bf.shape
    return lax.bitcast_convert_type(
        x_bf.reshape(shp[:-1] + (shp[-1] // 2, 2)), jnp.float32)


def _unpack_bf16_rows(x_f32):
    shp = x_f32.shape
    return lax.bitcast_convert_type(x_f32, jnp.bfloat16).reshape(
        shp[:-1] + (shp[-1] * 2,))


def kernel(x, embeddings, params):
    x2 = _run_front(x, embeddings, params)

    s1, s2, sg1, sg2, g1, g2, loss = _run_gating(x2, params['w_gating'])
    aux = loss.reshape(())

    scatter_k = _make_sc_scatter()
    xe = scatter_k(x2.reshape(B * C, L), s1.reshape(B * C), s2.reshape(B * C))

    eo = _run_experts(xe, params['moe_w1'], params['moe_w2'])

    gather_k = _make_sc_gather()
    a_f, b_f = gather_k(eo, sg1.reshape(B * C), sg2.reshape(B * C))

    conv_w3 = jnp.moveaxis(params['conv_w'], 2, 0)
    conv_b = params['conv_b'].reshape(2 * C, 1)
    y, xm = _run_tail(a_f.reshape(B, C, L), b_f.reshape(B, C, L), g1, g2,
                      conv_w3, conv_b)
    return (y, xm, aux)
